# baseline (speedup 1.0000x reference)
import functools

import jax
import jax.numpy as jnp
from jax import lax
from jax.experimental import pallas as pl
from jax.experimental.pallas import tpu as pltpu
from jax.experimental.pallas import tpu_sc as plsc

_NSC = 2
_NSUB = 16
_NW = _NSC * _NSUB
_MAXCW = 128
_BLK = 1024
_MBLK = 5000
_ACC_ALIGN = 2048


def _chunk_width(ew: int) -> int:
    for cw in range(_MAXCW - _MAXCW % 8, 0, -8):
        if ew % cw == 0:
            return cw
    return 0


@functools.lru_cache(maxsize=None)
def _degree_call_tiled(n_pad: int, tiles: int):
    slc = n_pad // _NSUB
    base_ch = tiles // _NW
    extra = tiles - base_ch * _NW
    ch_max = base_ch + (1 if extra else 0)
    mesh = plsc.VectorSubcoreMesh(core_axis_name="c", subcore_axis_name="s")

    @functools.partial(
        pl.kernel,
        out_type=jax.ShapeDtypeStruct((_NSC, n_pad), jnp.float32),
        mesh=mesh,
        scratch_types=[
            pltpu.VMEM((ch_max, _MAXCW), jnp.int32),
            pltpu.VMEM((_MAXCW,), jnp.float32),
            pltpu.VMEM((slc,), jnp.float32),
            pltpu.VMEM_SHARED((n_pad,), jnp.float32),
            pltpu.SemaphoreType.DMA,
            pltpu.SemaphoreType.DMA,
        ],
        compiler_params=pltpu.CompilerParams(use_tc_tiling_on_sc=True),
    )
    def deg_kernel(ei_hbm, out_hbm, idx_v, ones_v, zeros_v, acc_sh, sem_i, sem_s):
        c = lax.axis_index("c")
        s = lax.axis_index("s")
        w = c * _NSUB + s
        my_ch = base_ch + jnp.where(w < extra, 1, 0)

        def fire_loads(j, carry):
            t = j * _NW + w
            pltpu.async_copy(
                ei_hbm.at[1, pl.ds(t * _MAXCW, _MAXCW)], idx_v.at[j], sem_i
            )
            return carry

        lax.fori_loop(0, my_ch, fire_loads, 0)

        def fill_ones(j, carry):
            ones_v[pl.ds(j * 16, 16)] = jnp.full((16,), 1.0, jnp.float32)
            return carry

        lax.fori_loop(0, _MAXCW // 16, fill_ones, 0)

        def fill_zeros(j, carry):
            zeros_v[pl.ds(j * 16, 16)] = jnp.zeros((16,), jnp.float32)
            return carry

        lax.fori_loop(0, slc // 16, fill_zeros, 0)

        pltpu.sync_copy(zeros_v, acc_sh.at[pl.ds(s * slc, slc)])

        def drain_loads(j, carry):
            pltpu.make_async_copy(
                ei_hbm.at[1, pl.ds(0, _MAXCW)], idx_v.at[0], sem_i
            ).wait()
            return carry

        lax.fori_loop(0, my_ch, drain_loads, 0)
        plsc.subcore_barrier()

        def fire(j, carry):
            pltpu.async_copy(ones_v, acc_sh.at[idx_v.at[j]], sem_s, add=True)
            return carry

        lax.fori_loop(0, my_ch, fire, 0)

        def drain(j, carry):
            pltpu.make_async_copy(ones_v, acc_sh.at[idx_v.at[0]], sem_s).wait()
            return carry

        lax.fori_loop(0, my_ch, drain, 0)
        plsc.subcore_barrier()

        pltpu.sync_copy(
            acc_sh.at[pl.ds(s * slc, slc)],
            out_hbm.at[c, pl.ds(s * slc, slc)],
        )

    return deg_kernel


@functools.lru_cache(maxsize=None)
def _degree_call(n_pad: int, ch: int, cw: int):
    slc = n_pad // _NSUB
    mesh = plsc.VectorSubcoreMesh(core_axis_name="c", subcore_axis_name="s")

    @functools.partial(
        pl.kernel,
        out_type=jax.ShapeDtypeStruct((_NSC, n_pad), jnp.float32),
        mesh=mesh,
        scratch_types=[
            pltpu.VMEM((ch, cw), jnp.int32),
            pltpu.VMEM((cw,), jnp.float32),
            pltpu.VMEM((slc,), jnp.float32),
            pltpu.VMEM_SHARED((n_pad,), jnp.float32),
            pltpu.SemaphoreType.DMA,
        ],
    )
    def deg_kernel(dst_hbm, out_hbm, idx_v, ones_v, zeros_v, acc_sh, sem):
        c = lax.axis_index("c")
        s = lax.axis_index("s")
        w = c * _NSUB + s

        idx_cp = pltpu.async_copy(dst_hbm.at[w], idx_v, sem)

        def fill_ones(j, carry):
            ones_v[pl.ds(j * 16, 16)] = jnp.full((16,), 1.0, jnp.float32)
            return carry

        lax.fori_loop(0, cw // 16, fill_ones, 0)

        def fill_zeros(j, carry):
            zeros_v[pl.ds(j * 16, 16)] = jnp.zeros((16,), jnp.float32)
            return carry

        lax.fori_loop(0, slc // 16, fill_zeros, 0)

        pltpu.sync_copy(zeros_v, acc_sh.at[pl.ds(s * slc, slc)])
        idx_cp.wait()
        plsc.subcore_barrier()

        def fire(j, carry):
            pltpu.async_copy(ones_v, acc_sh.at[idx_v.at[j]], sem, add=True)
            return carry

        lax.fori_loop(0, ch, fire, 0)

        def drain(j, carry):
            pltpu.make_async_copy(ones_v, acc_sh.at[idx_v.at[0]], sem).wait()
            return carry

        lax.fori_loop(0, ch, drain, 0)
        plsc.subcore_barrier()

        pltpu.sync_copy(
            acc_sh.at[pl.ds(s * slc, slc)],
            out_hbm.at[c, pl.ds(s * slc, slc)],
        )

    return deg_kernel


def _matmul_body(x_ref, w0_ref, w1_ref, out_ref):
    cdims = (((1,), (1,)), ((), ()))
    h = jnp.maximum(
        lax.dot_general(
            x_ref[...], w0_ref[...], cdims, preferred_element_type=jnp.float32
        ),
        0.0,
    )
    out_ref[...] = lax.dot_general(
        w1_ref[...], h, cdims, preferred_element_type=jnp.float32
    )


def _mask_body(o_ref, deg_ref, out_ref):
    n = o_ref.shape[1]
    deg = deg_ref[...]
    d = deg[0:1, :n] + deg[1:2, :n]
    ind = (d > 0.0).astype(jnp.float32)
    out_ref[...] = o_ref[...] * ind


def kernel(x, edge_index, W0, A0, W1, A1):
    del A0, A1
    n, in_dim = x.shape
    heads, hid, _ = W0.shape
    ncls = W1.shape[0]
    e = edge_index.shape[1]

    n_pad = -(-n // _ACC_ALIGN) * _ACC_ALIGN
    if e % _MAXCW == 0:
        deg = _degree_call_tiled(n_pad, e // _MAXCW)(edge_index)
    else:
        dst = edge_index[1]
        cw = _MAXCW
        ch = -(-e // (_NW * cw))
        dst3d = jnp.concatenate(
            [dst, jnp.full((_NW * ch * cw - e,), n, jnp.int32)]
        ).reshape(_NW, ch, cw)
        deg = _degree_call(n_pad, ch, cw)(dst3d)

    w0cat = W0.reshape(heads * hid, in_dim)
    o = pl.pallas_call(
        _matmul_body,
        grid=(pl.cdiv(n, _BLK),),
        in_specs=[
            pl.BlockSpec((_BLK, in_dim), lambda i: (i, 0)),
            pl.BlockSpec((heads * hid, in_dim), lambda i: (0, 0)),
            pl.BlockSpec((ncls, heads * hid), lambda i: (0, 0)),
        ],
        out_specs=pl.BlockSpec((ncls, _BLK), lambda i: (0, i)),
        out_shape=jax.ShapeDtypeStruct((ncls, n), jnp.float32),
    )(x, w0cat, W1)

    masked = pl.pallas_call(
        _mask_body,
        out_shape=jax.ShapeDtypeStruct((ncls, n), jnp.float32),
        input_output_aliases={0: 0},
    )(o, deg)
    return jnp.transpose(masked)


# speedup vs baseline: 1.0759x; 1.0759x over previous
import functools

import jax
import jax.numpy as jnp
from jax import lax
from jax.experimental import pallas as pl
from jax.experimental.pallas import tpu as pltpu
from jax.experimental.pallas import tpu_sc as plsc

_NSC = 2
_NSUB = 16
_NW = _NSC * _NSUB
_MAXCW = 128
_BLK = 2560
_MBLK = 5000
_ACC_ALIGN = 2048


def _chunk_width(ew: int) -> int:
    for cw in range(_MAXCW - _MAXCW % 8, 0, -8):
        if ew % cw == 0:
            return cw
    return 0


@functools.lru_cache(maxsize=None)
def _degree_call_tiled(n_pad: int, tiles: int):
    slc = n_pad // _NSUB
    base_ch = tiles // _NW
    extra = tiles - base_ch * _NW
    ch_max = base_ch + (1 if extra else 0)
    mesh = plsc.VectorSubcoreMesh(core_axis_name="c", subcore_axis_name="s")

    @functools.partial(
        pl.kernel,
        out_type=jax.ShapeDtypeStruct((_NSC, n_pad), jnp.float32),
        mesh=mesh,
        scratch_types=[
            pltpu.VMEM((ch_max, _MAXCW), jnp.int32),
            pltpu.VMEM((_MAXCW,), jnp.float32),
            pltpu.VMEM((slc,), jnp.float32),
            pltpu.VMEM_SHARED((n_pad,), jnp.float32),
            pltpu.SemaphoreType.DMA,
            pltpu.SemaphoreType.DMA,
        ],
        compiler_params=pltpu.CompilerParams(use_tc_tiling_on_sc=True),
    )
    def deg_kernel(ei_hbm, out_hbm, idx_v, ones_v, zeros_v, acc_sh, sem_i, sem_s):
        c = lax.axis_index("c")
        s = lax.axis_index("s")
        w = c * _NSUB + s
        my_ch = base_ch + jnp.where(w < extra, 1, 0)

        def fire_loads(j, carry):
            t = j * _NW + w
            pltpu.async_copy(
                ei_hbm.at[1, pl.ds(t * _MAXCW, _MAXCW)], idx_v.at[j], sem_i
            )
            return carry

        lax.fori_loop(0, my_ch, fire_loads, 0)

        def fill_ones(j, carry):
            ones_v[pl.ds(j * 16, 16)] = jnp.full((16,), 1.0, jnp.float32)
            return carry

        lax.fori_loop(0, _MAXCW // 16, fill_ones, 0)

        def fill_zeros(j, carry):
            zeros_v[pl.ds(j * 16, 16)] = jnp.zeros((16,), jnp.float32)
            return carry

        lax.fori_loop(0, slc // 16, fill_zeros, 0)

        pltpu.sync_copy(zeros_v, acc_sh.at[pl.ds(s * slc, slc)])

        def drain_loads(j, carry):
            pltpu.make_async_copy(
                ei_hbm.at[1, pl.ds(0, _MAXCW)], idx_v.at[0], sem_i
            ).wait()
            return carry

        lax.fori_loop(0, my_ch, drain_loads, 0)
        plsc.subcore_barrier()

        def fire(j, carry):
            pltpu.async_copy(ones_v, acc_sh.at[idx_v.at[j]], sem_s, add=True)
            return carry

        lax.fori_loop(0, my_ch, fire, 0)

        def drain(j, carry):
            pltpu.make_async_copy(ones_v, acc_sh.at[idx_v.at[0]], sem_s).wait()
            return carry

        lax.fori_loop(0, my_ch, drain, 0)
        plsc.subcore_barrier()

        pltpu.sync_copy(
            acc_sh.at[pl.ds(s * slc, slc)],
            out_hbm.at[c, pl.ds(s * slc, slc)],
        )

    return deg_kernel


@functools.lru_cache(maxsize=None)
def _degree_call(n_pad: int, ch: int, cw: int):
    slc = n_pad // _NSUB
    mesh = plsc.VectorSubcoreMesh(core_axis_name="c", subcore_axis_name="s")

    @functools.partial(
        pl.kernel,
        out_type=jax.ShapeDtypeStruct((_NSC, n_pad), jnp.float32),
        mesh=mesh,
        scratch_types=[
            pltpu.VMEM((ch, cw), jnp.int32),
            pltpu.VMEM((cw,), jnp.float32),
            pltpu.VMEM((slc,), jnp.float32),
            pltpu.VMEM_SHARED((n_pad,), jnp.float32),
            pltpu.SemaphoreType.DMA,
        ],
    )
    def deg_kernel(dst_hbm, out_hbm, idx_v, ones_v, zeros_v, acc_sh, sem):
        c = lax.axis_index("c")
        s = lax.axis_index("s")
        w = c * _NSUB + s

        idx_cp = pltpu.async_copy(dst_hbm.at[w], idx_v, sem)

        def fill_ones(j, carry):
            ones_v[pl.ds(j * 16, 16)] = jnp.full((16,), 1.0, jnp.float32)
            return carry

        lax.fori_loop(0, cw // 16, fill_ones, 0)

        def fill_zeros(j, carry):
            zeros_v[pl.ds(j * 16, 16)] = jnp.zeros((16,), jnp.float32)
            return carry

        lax.fori_loop(0, slc // 16, fill_zeros, 0)

        pltpu.sync_copy(zeros_v, acc_sh.at[pl.ds(s * slc, slc)])
        idx_cp.wait()
        plsc.subcore_barrier()

        def fire(j, carry):
            pltpu.async_copy(ones_v, acc_sh.at[idx_v.at[j]], sem, add=True)
            return carry

        lax.fori_loop(0, ch, fire, 0)

        def drain(j, carry):
            pltpu.make_async_copy(ones_v, acc_sh.at[idx_v.at[0]], sem).wait()
            return carry

        lax.fori_loop(0, ch, drain, 0)
        plsc.subcore_barrier()

        pltpu.sync_copy(
            acc_sh.at[pl.ds(s * slc, slc)],
            out_hbm.at[c, pl.ds(s * slc, slc)],
        )

    return deg_kernel


def _matmul_body(x_ref, w0_ref, w1_ref, out_ref):
    cdims = (((1,), (1,)), ((), ()))
    h = jnp.maximum(
        lax.dot_general(
            x_ref[...], w0_ref[...], cdims, preferred_element_type=jnp.float32
        ),
        0.0,
    )
    out_ref[...] = lax.dot_general(
        w1_ref[...], h, cdims, preferred_element_type=jnp.float32
    )


def _mask_body(o_ref, deg_ref, out_ref):
    n = o_ref.shape[1]
    deg = deg_ref[...]
    d = deg[0:1, :n] + deg[1:2, :n]
    ind = (d > 0.0).astype(jnp.float32)
    out_ref[...] = o_ref[...] * ind


def kernel(x, edge_index, W0, A0, W1, A1):
    del A0, A1
    n, in_dim = x.shape
    heads, hid, _ = W0.shape
    ncls = W1.shape[0]
    e = edge_index.shape[1]

    n_pad = -(-n // _ACC_ALIGN) * _ACC_ALIGN
    if e % _MAXCW == 0:
        deg = _degree_call_tiled(n_pad, e // _MAXCW)(edge_index)
    else:
        dst = edge_index[1]
        cw = _MAXCW
        ch = -(-e // (_NW * cw))
        dst3d = jnp.concatenate(
            [dst, jnp.full((_NW * ch * cw - e,), n, jnp.int32)]
        ).reshape(_NW, ch, cw)
        deg = _degree_call(n_pad, ch, cw)(dst3d)

    w0cat = W0.reshape(heads * hid, in_dim)
    o = pl.pallas_call(
        _matmul_body,
        grid=(pl.cdiv(n, _BLK),),
        in_specs=[
            pl.BlockSpec((_BLK, in_dim), lambda i: (i, 0)),
            pl.BlockSpec((heads * hid, in_dim), lambda i: (0, 0)),
            pl.BlockSpec((ncls, heads * hid), lambda i: (0, 0)),
        ],
        out_specs=pl.BlockSpec((ncls, _BLK), lambda i: (0, i)),
        out_shape=jax.ShapeDtypeStruct((ncls, n), jnp.float32),
    )(x, w0cat, W1)

    masked = pl.pallas_call(
        _mask_body,
        out_shape=jax.ShapeDtypeStruct((ncls, n), jnp.float32),
        input_output_aliases={0: 0},
    )(o, deg)
    return jnp.transpose(masked)


# speedup vs baseline: 1.0825x; 1.0061x over previous
import functools

import jax
import jax.numpy as jnp
from jax import lax
from jax.experimental import pallas as pl
from jax.experimental.pallas import tpu as pltpu
from jax.experimental.pallas import tpu_sc as plsc

_NSC = 2
_NSUB = 16
_NW = _NSC * _NSUB
_MAXCW = 128
_BLK = 2048
_MBLK = 5000
_ACC_ALIGN = 2048


def _chunk_width(ew: int) -> int:
    for cw in range(_MAXCW - _MAXCW % 8, 0, -8):
        if ew % cw == 0:
            return cw
    return 0


@functools.lru_cache(maxsize=None)
def _degree_call_tiled(n_pad: int, tiles: int):
    slc = n_pad // _NSUB
    base_ch = tiles // _NW
    extra = tiles - base_ch * _NW
    ch_max = base_ch + (1 if extra else 0)
    mesh = plsc.VectorSubcoreMesh(core_axis_name="c", subcore_axis_name="s")

    @functools.partial(
        pl.kernel,
        out_type=jax.ShapeDtypeStruct((_NSC, n_pad), jnp.float32),
        mesh=mesh,
        scratch_types=[
            pltpu.VMEM((ch_max, _MAXCW), jnp.int32),
            pltpu.VMEM((_MAXCW,), jnp.float32),
            pltpu.VMEM((slc,), jnp.float32),
            pltpu.VMEM_SHARED((n_pad,), jnp.float32),
            pltpu.SemaphoreType.DMA,
            pltpu.SemaphoreType.DMA,
        ],
        compiler_params=pltpu.CompilerParams(use_tc_tiling_on_sc=True),
    )
    def deg_kernel(ei_hbm, out_hbm, idx_v, ones_v, zeros_v, acc_sh, sem_i, sem_s):
        c = lax.axis_index("c")
        s = lax.axis_index("s")
        w = c * _NSUB + s
        my_ch = base_ch + jnp.where(w < extra, 1, 0)

        def fire_loads(j, carry):
            t = j * _NW + w
            pltpu.async_copy(
                ei_hbm.at[1, pl.ds(t * _MAXCW, _MAXCW)], idx_v.at[j], sem_i
            )
            return carry

        lax.fori_loop(0, my_ch, fire_loads, 0)

        def fill_ones(j, carry):
            ones_v[pl.ds(j * 16, 16)] = jnp.full((16,), 1.0, jnp.float32)
            return carry

        lax.fori_loop(0, _MAXCW // 16, fill_ones, 0)

        def fill_zeros(j, carry):
            zeros_v[pl.ds(j * 16, 16)] = jnp.zeros((16,), jnp.float32)
            return carry

        lax.fori_loop(0, slc // 16, fill_zeros, 0)

        pltpu.sync_copy(zeros_v, acc_sh.at[pl.ds(s * slc, slc)])

        def drain_loads(j, carry):
            pltpu.make_async_copy(
                ei_hbm.at[1, pl.ds(0, _MAXCW)], idx_v.at[0], sem_i
            ).wait()
            return carry

        lax.fori_loop(0, my_ch, drain_loads, 0)
        plsc.subcore_barrier()

        def fire(j, carry):
            pltpu.async_copy(ones_v, acc_sh.at[idx_v.at[j]], sem_s, add=True)
            return carry

        lax.fori_loop(0, my_ch, fire, 0)

        def drain(j, carry):
            pltpu.make_async_copy(ones_v, acc_sh.at[idx_v.at[0]], sem_s).wait()
            return carry

        lax.fori_loop(0, my_ch, drain, 0)
        plsc.subcore_barrier()

        pltpu.sync_copy(
            acc_sh.at[pl.ds(s * slc, slc)],
            out_hbm.at[c, pl.ds(s * slc, slc)],
        )

    return deg_kernel


@functools.lru_cache(maxsize=None)
def _degree_call(n_pad: int, ch: int, cw: int):
    slc = n_pad // _NSUB
    mesh = plsc.VectorSubcoreMesh(core_axis_name="c", subcore_axis_name="s")

    @functools.partial(
        pl.kernel,
        out_type=jax.ShapeDtypeStruct((_NSC, n_pad), jnp.float32),
        mesh=mesh,
        scratch_types=[
            pltpu.VMEM((ch, cw), jnp.int32),
            pltpu.VMEM((cw,), jnp.float32),
            pltpu.VMEM((slc,), jnp.float32),
            pltpu.VMEM_SHARED((n_pad,), jnp.float32),
            pltpu.SemaphoreType.DMA,
        ],
    )
    def deg_kernel(dst_hbm, out_hbm, idx_v, ones_v, zeros_v, acc_sh, sem):
        c = lax.axis_index("c")
        s = lax.axis_index("s")
        w = c * _NSUB + s

        idx_cp = pltpu.async_copy(dst_hbm.at[w], idx_v, sem)

        def fill_ones(j, carry):
            ones_v[pl.ds(j * 16, 16)] = jnp.full((16,), 1.0, jnp.float32)
            return carry

        lax.fori_loop(0, cw // 16, fill_ones, 0)

        def fill_zeros(j, carry):
            zeros_v[pl.ds(j * 16, 16)] = jnp.zeros((16,), jnp.float32)
            return carry

        lax.fori_loop(0, slc // 16, fill_zeros, 0)

        pltpu.sync_copy(zeros_v, acc_sh.at[pl.ds(s * slc, slc)])
        idx_cp.wait()
        plsc.subcore_barrier()

        def fire(j, carry):
            pltpu.async_copy(ones_v, acc_sh.at[idx_v.at[j]], sem, add=True)
            return carry

        lax.fori_loop(0, ch, fire, 0)

        def drain(j, carry):
            pltpu.make_async_copy(ones_v, acc_sh.at[idx_v.at[0]], sem).wait()
            return carry

        lax.fori_loop(0, ch, drain, 0)
        plsc.subcore_barrier()

        pltpu.sync_copy(
            acc_sh.at[pl.ds(s * slc, slc)],
            out_hbm.at[c, pl.ds(s * slc, slc)],
        )

    return deg_kernel


def _matmul_body(x_ref, w0_ref, w1_ref, out_ref):
    cdims = (((1,), (1,)), ((), ()))
    h = jnp.maximum(
        lax.dot_general(
            x_ref[...], w0_ref[...], cdims, preferred_element_type=jnp.float32
        ),
        0.0,
    )
    out_ref[...] = lax.dot_general(
        w1_ref[...], h, cdims, preferred_element_type=jnp.float32
    )


def _mask_body(o_ref, deg_ref, out_ref):
    n = o_ref.shape[1]
    deg = deg_ref[...]
    d = deg[0:1, :n] + deg[1:2, :n]
    ind = (d > 0.0).astype(jnp.float32)
    out_ref[...] = o_ref[...] * ind


def kernel(x, edge_index, W0, A0, W1, A1):
    del A0, A1
    n, in_dim = x.shape
    heads, hid, _ = W0.shape
    ncls = W1.shape[0]
    e = edge_index.shape[1]

    n_pad = -(-n // _ACC_ALIGN) * _ACC_ALIGN
    w0cat = W0.reshape(heads * hid, in_dim)
    o = pl.pallas_call(
        _matmul_body,
        grid=(pl.cdiv(n, _BLK),),
        in_specs=[
            pl.BlockSpec((_BLK, in_dim), lambda i: (i, 0)),
            pl.BlockSpec((heads * hid, in_dim), lambda i: (0, 0)),
            pl.BlockSpec((ncls, heads * hid), lambda i: (0, 0)),
        ],
        out_specs=pl.BlockSpec((ncls, _BLK), lambda i: (0, i)),
        out_shape=jax.ShapeDtypeStruct((ncls, n), jnp.float32),
    )(x, w0cat, W1)

    if e % _MAXCW == 0:
        deg = _degree_call_tiled(n_pad, e // _MAXCW)(edge_index)
    else:
        dst = edge_index[1]
        cw = _MAXCW
        ch = -(-e // (_NW * cw))
        dst3d = jnp.concatenate(
            [dst, jnp.full((_NW * ch * cw - e,), n, jnp.int32)]
        ).reshape(_NW, ch, cw)
        deg = _degree_call(n_pad, ch, cw)(dst3d)

    masked = pl.pallas_call(
        _mask_body,
        out_shape=jax.ShapeDtypeStruct((ncls, n), jnp.float32),
        input_output_aliases={0: 0},
    )(o, deg)
    return jnp.transpose(masked)
